# baseline (device time: 98947 ns/iter reference)
import functools

import jax
import jax.numpy as jnp
from jax import lax
from jax.experimental import pallas as pl
from jax.experimental.pallas import tpu as pltpu

N_DEV = 4
N_SUB = 2


def kernel(A, B):
    m, k = A.shape
    _, n = B.shape
    mc = m // N_DEV
    sub = mc // N_SUB
    half = n // 2

    f32 = jnp.float32
    bf16 = jnp.bfloat16

    def body(a_ref, b_ref, out_ref, rs0_buf, rs_buf, ag_buf,
             rs_send_sems, rs_recv_sems, ag_send_sems, ag_recv_sems):
        my = lax.axis_index("i")
        left = (my + N_DEV - 1) % N_DEV
        right = (my + 1) % N_DEV
        ring_dst = (right, left)

        barrier_sem = pltpu.get_barrier_semaphore()
        for nbr in (left, right):
            pl.semaphore_signal(
                barrier_sem, inc=1,
                device_id=(nbr,), device_id_type=pl.DeviceIdType.MESH,
            )
        pl.semaphore_wait(barrier_sem, 2)

        def srows(c, u):
            return pl.ds(c * mc + u * sub, sub)

        cols = (slice(0, half), slice(half, n))

        def half_dot(c, u, r):
            return jnp.dot(
                a_ref[srows(c, u), :], b_ref[:, cols[r]],
                preferred_element_type=f32,
            )

        def store_partial(c, u, r):
            out_ref[srows(c, u), cols[r]] = half_dot(c, u, r)

        def make_rs(s, u, r):
            src = rs0_buf.at[r] if s == 0 else rs_buf.at[r, s - 1]
            return pltpu.make_async_remote_copy(
                src_ref=src.at[pl.ds(u * sub, sub), :],
                dst_ref=rs_buf.at[r, s, pl.ds(u * sub, sub), :],
                send_sem=rs_send_sems.at[r, s, u],
                recv_sem=rs_recv_sems.at[r, s, u],
                device_id=(ring_dst[r],),
                device_id_type=pl.DeviceIdType.MESH,
            )

        def make_ag(t, u, r):
            return pltpu.make_async_remote_copy(
                src_ref=ag_buf.at[r, t, pl.ds(u * sub, sub), :],
                dst_ref=ag_buf.at[r, t + 1, pl.ds(u * sub, sub), :],
                send_sem=ag_send_sems.at[r, t, u],
                recv_sem=ag_recv_sems.at[r, t, u],
                device_id=(ring_dst[r],),
                device_id_type=pl.DeviceIdType.MESH,
            )

        rs = [[[make_rs(s, u, r) for r in range(2)] for u in range(N_SUB)]
              for s in range(N_DEV - 1)]
        ag = [[[make_ag(t, u, r) for r in range(2)] for u in range(N_SUB)]
              for t in range(N_DEV - 1)]

        for u in range(N_SUB):
            for r in range(2):
                rs0_buf[r, pl.ds(u * sub, sub), :] = (
                    half_dot(my, u, r).astype(bf16)
                )
                rs[0][u][r].start()

        rs_c = [((my + N_DEV - 1 - s) % N_DEV, (my + 1 + s) % N_DEV)
                for s in range(N_DEV - 1)]

        def acc(s, u):
            for r in range(2):
                rs_buf[r, s, pl.ds(u * sub, sub), :] = (
                    rs_buf[r, s, pl.ds(u * sub, sub), :].astype(f32)
                    + out_ref[srows(rs_c[s][r], u), cols[r]]
                ).astype(bf16)
                rs[s + 1][u][r].start()

        def final_acc(u):
            s = N_DEV - 2
            for r in range(2):
                a = (
                    rs_buf[r, s, pl.ds(u * sub, sub), :].astype(f32)
                    + out_ref[srows(rs_c[s][r], u), cols[r]]
                )
                out_ref[srows(rs_c[s][r], u), cols[r]] = a
                ag_buf[r, 0, pl.ds(u * sub, sub), :] = a.astype(bf16)
                ag[0][u][r].start()

        windows = [(0, 0), (0, 1), (1, 0), (1, 1), (2, 0), (2, 1)]
        for s, u in windows:
            for r in range(2):
                store_partial(rs_c[s][r], u, r)
            for r in range(2):
                rs[s][u][r].wait_recv()
            if s < N_DEV - 2:
                acc(s, u)
            else:
                final_acc(u)

        for t in range(N_DEV - 1):
            c = ((my + N_DEV - t) % N_DEV, (my + t) % N_DEV)
            for u in range(N_SUB):
                for r in range(2):
                    ag[t][u][r].wait_recv()
                if t < N_DEV - 2:
                    for r in range(2):
                        ag[t + 1][u][r].start()
                for r in range(2):
                    out_ref[srows(c[r], u), cols[r]] = (
                        ag_buf[r, t + 1, pl.ds(u * sub, sub), :].astype(f32)
                    )

        for group in (rs, ag):
            for hop in group:
                for u_list in hop:
                    for rdma in u_list:
                        rdma.wait_send()

        @functools.partial(
            pl.run_scoped, second_barrier=pltpu.SemaphoreType.REGULAR
        )
        def _(second_barrier):
            for nbr in (left, right):
                pl.semaphore_signal(
                    second_barrier, inc=1,
                    device_id=(nbr,), device_id_type=pl.DeviceIdType.MESH,
                )
            pl.semaphore_wait(second_barrier, 2)

    return pl.pallas_call(
        body,
        out_shape=jax.ShapeDtypeStruct((m, n), f32),
        in_specs=[
            pl.BlockSpec(memory_space=pltpu.VMEM),
            pl.BlockSpec(memory_space=pltpu.VMEM),
        ],
        out_specs=pl.BlockSpec(memory_space=pltpu.VMEM),
        scratch_shapes=[
            pltpu.VMEM((2, mc, half), bf16),
            pltpu.VMEM((2, N_DEV - 1, mc, half), bf16),
            pltpu.VMEM((2, N_DEV, mc, half), bf16),
            pltpu.SemaphoreType.DMA((2, N_DEV - 1, N_SUB)),
            pltpu.SemaphoreType.DMA((2, N_DEV - 1, N_SUB)),
            pltpu.SemaphoreType.DMA((2, N_DEV - 1, N_SUB)),
            pltpu.SemaphoreType.DMA((2, N_DEV - 1, N_SUB)),
        ],
        compiler_params=pltpu.CompilerParams(
            collective_id=0, vmem_limit_bytes=100 * 1024 * 1024
        ),
    )(A, B)
